# Initial kernel scaffold; baseline (speedup 1.0000x reference)
import jax
import jax.numpy as jnp
from jax.experimental import pallas as pl


def kernel(x, edge_attr, enc_n_W, enc_n_b, enc_e_W0, enc_e_b0, enc_e_W, enc_e_b, conv_W1, conv_b1, conv_W2, conv_W3, conv_b3, mlp_W1, mlp_b1, mlp_g, mlp_beta, mlp_W2, mlp_b2, out_W, out_b, out_Wf, out_bf, edge_index, batch, ptr):
    raise NotImplementedError("write your pallas kernel here")


# speedup vs baseline: 7.0494x; 7.0494x over previous
import functools

import jax
import jax.numpy as jnp
from jax import lax
from jax.experimental import pallas as pl
from jax.experimental.pallas import tpu as pltpu
from jax.experimental.pallas import tpu_sc as plsc

N = 10000
E = 320000
D = 128
DM = 256
NG = 16
NC, NS = 2, 16
TILES = NC * NS
EPT = E // TILES
CH = 80
NCHUNK = EPT // CH
NPA = 10240
RPT = NPA // NS
RB = 1000
NB = N // RB
NPAD = 10240

@functools.cache
def _sc_kernels():
    mesh = plsc.VectorSubcoreMesh(
        core_axis_name="c", subcore_axis_name="s",
        num_cores=NC, num_subcores=NS)

    @functools.partial(
        pl.kernel,
        out_type=jax.ShapeDtypeStruct((NC, NPA, D), jnp.float32),
        mesh=mesh,
        scratch_types=[
            pltpu.VMEM_SHARED((NPA, D), jnp.float32),
            pltpu.VMEM((NCHUNK, CH), jnp.int32),
            pltpu.VMEM((NCHUNK, CH), jnp.int32),
            pltpu.VMEM((CH, D), jnp.float32),
            pltpu.SemaphoreType.DMA,
        ],
    )
    def _spmm(x_hbm, src_hbm, dst_hbm, zeros_hbm, out_hbm,
              acc, src_v, dst_v, rows_v, sem):
        c = lax.axis_index("c")
        s = lax.axis_index("s")
        t = c * NS + s
        pltpu.sync_copy(zeros_hbm, acc.at[pl.ds(s * RPT, RPT)])
        pltpu.sync_copy(src_hbm.at[t], src_v)
        pltpu.sync_copy(dst_hbm.at[t], dst_v)
        plsc.subcore_barrier()

        def body(j, carry):
            pltpu.async_copy(x_hbm.at[src_v.at[j]], rows_v, sem).wait()
            pltpu.sync_copy(rows_v, acc.at[dst_v.at[j]], add=True)
            return carry

        lax.fori_loop(0, NCHUNK, body, 0)
        plsc.subcore_barrier()
        pltpu.sync_copy(acc.at[pl.ds(s * RPT, RPT)],
                        out_hbm.at[c, pl.ds(s * RPT, RPT)])

    return _spmm


def _leaky(v):
    return jnp.where(v > 0, v, 0.01 * v)


def _enc_body(x_ref, w_ref, b_ref, o_ref):
    y = x_ref[...]
    for i in range(3):
        y = _leaky(jnp.dot(y, w_ref[i], preferred_element_type=jnp.float32)
                   + b_ref[i])
    o_ref[...] = y


def _make_encode(interpret=False):
    return pl.pallas_call(
        _enc_body,
        grid=(NB,),
        in_specs=[
            pl.BlockSpec((RB, D), lambda j: (j, 0)),
            pl.BlockSpec((3, D, D), lambda j: (0, 0, 0)),
            pl.BlockSpec((3, 1, D), lambda j: (0, 0, 0)),
        ],
        out_specs=pl.BlockSpec((RB, D), lambda j: (j, 0)),
        out_shape=jax.ShapeDtypeStruct((N, D), jnp.float32),
        interpret=interpret,
    )


_encode = _make_encode()


def _layer_body(g0, g1, x, x0, d0, d1, w1c, b1c, w2c, w3c, b3c,
                w1m, b1m, gam, bet, w2m, b2m, o_ref, u_s, stat_s,
                *, with_skip):
    p = pl.program_id(0)
    j = pl.program_id(1)

    @pl.when(p == 0)
    def _compute():
        @pl.when(j == 0)
        def _init():
            stat_s[...] = jnp.zeros_like(stat_s)

        deg = (d0[...] + d1[...])[:, :1]
        g = g0[...] + g1[...]
        xb = x[...]
        h = (jnp.dot(g, w1c[...], preferred_element_type=jnp.float32)
             + deg * b1c[...]
             - deg * jnp.dot(xb, w2c[...], preferred_element_type=jnp.float32)
             + jnp.dot(xb, w3c[...], preferred_element_type=jnp.float32)
             + b3c[...])
        u = jnp.dot(h, w1m[...], preferred_element_type=jnp.float32) + b1m[...]
        u_s[pl.ds(j * RB, RB), :] = u
        stat_s[0:1, :] += jnp.sum(u, axis=0, keepdims=True)
        stat_s[1:2, :] += jnp.sum(u * u, axis=0, keepdims=True)

    @pl.when(p == 1)
    def _normalize():
        @pl.when(j == 0)
        def _stats():
            mean = stat_s[0:1, :] * (1.0 / N)
            var = stat_s[1:2, :] * (1.0 / N) - mean * mean
            scale = gam[...] * lax.rsqrt(var + 1e-5)
            stat_s[2:3, :] = scale
            stat_s[3:4, :] = bet[...] - mean * scale

        u = u_s[pl.ds(j * RB, RB), :]
        v = _leaky(u * stat_s[2:3, :] + stat_s[3:4, :])
        o = jnp.dot(v, w2m[...], preferred_element_type=jnp.float32) + b2m[...]
        if with_skip:
            o = o + x0[...]
        o_ref[...] = o


def _make_layer(with_skip, interpret=False):
    return pl.pallas_call(
        functools.partial(_layer_body, with_skip=with_skip),
        grid=(2, NB),
        in_specs=[
            pl.BlockSpec((RB, D), lambda p, j: (j, 0)),
            pl.BlockSpec((RB, D), lambda p, j: (j, 0)),
            pl.BlockSpec((RB, D), lambda p, j: (j, 0)),
            pl.BlockSpec((RB, D), lambda p, j: (j, 0)),
            pl.BlockSpec((RB, D), lambda p, j: (j, 0)),
            pl.BlockSpec((RB, D), lambda p, j: (j, 0)),
            pl.BlockSpec((D, D), lambda p, j: (0, 0)),
            pl.BlockSpec((1, D), lambda p, j: (0, 0)),
            pl.BlockSpec((D, D), lambda p, j: (0, 0)),
            pl.BlockSpec((D, D), lambda p, j: (0, 0)),
            pl.BlockSpec((1, D), lambda p, j: (0, 0)),
            pl.BlockSpec((D, DM), lambda p, j: (0, 0)),
            pl.BlockSpec((1, DM), lambda p, j: (0, 0)),
            pl.BlockSpec((1, DM), lambda p, j: (0, 0)),
            pl.BlockSpec((1, DM), lambda p, j: (0, 0)),
            pl.BlockSpec((DM, D), lambda p, j: (0, 0)),
            pl.BlockSpec((1, D), lambda p, j: (0, 0)),
        ],
        out_specs=pl.BlockSpec((RB, D), lambda p, j: (j, 0)),
        out_shape=jax.ShapeDtypeStruct((N, D), jnp.float32),
        scratch_shapes=[
            pltpu.VMEM((N, DM), jnp.float32),
            pltpu.VMEM((8, DM), jnp.float32),
        ],
        interpret=interpret,
    )


_layer_skip = _make_layer(True)
_layer_noskip = _make_layer(False)


def _pool_body(x_ref, b_ref, w_ref, bo_ref, wf_ref, bf_ref, o_ref):
    bt = b_ref[...]
    gid = lax.broadcasted_iota(jnp.int32, (NG, NPAD), 0)
    oh = jnp.where(gid == bt, 1.0, 0.0)
    cnt = jnp.sum(oh, axis=1, keepdims=True)
    pooled = jnp.dot(oh, x_ref[...], preferred_element_type=jnp.float32)
    o = pooled / jnp.maximum(cnt, 1.0)
    for i in range(2):
        o = _leaky(jnp.dot(o, w_ref[i], preferred_element_type=jnp.float32)
                   + bo_ref[i])
    o_ref[...] = (jnp.dot(o, wf_ref[...], preferred_element_type=jnp.float32)
                  + bf_ref[...])


def _make_pool(interpret=False):
    return pl.pallas_call(
        _pool_body,
        out_shape=jax.ShapeDtypeStruct((NG, 1), jnp.float32),
        interpret=interpret,
    )


_pool_head = _make_pool()


def kernel(x, edge_attr, enc_n_W, enc_n_b, enc_e_W0, enc_e_b0, enc_e_W,
           enc_e_b, conv_W1, conv_b1, conv_W2, conv_W3, conv_b3,
           mlp_W1, mlp_b1, mlp_g, mlp_beta, mlp_W2, mlp_b2,
           out_W, out_b, out_Wf, out_bf, edge_index, batch, ptr):
    f32 = jnp.float32
    src3d = edge_index[0].reshape(TILES, NCHUNK, CH)
    dst3d = edge_index[1].reshape(TILES, NCHUNK, CH)
    zero_rows = jnp.zeros((RPT, D), f32)

    spmm = _sc_kernels()
    xe = _encode(x, enc_n_W, enc_n_b.reshape(3, 1, D))

    ones_tab = jnp.ones((N, D), f32)
    degp = spmm(ones_tab, dst3d, dst3d, zero_rows)
    d0, d1 = degp[0], degp[1]

    x0 = xe
    xc = xe
    for i in range(3):
        gp = spmm(xc, src3d, dst3d, zero_rows)
        layer = _layer_skip if i == 0 else _layer_noskip
        xc = layer(
            gp[0], gp[1], xc, x0, d0, d1,
            conv_W1[i], conv_b1[i].reshape(1, D), conv_W2[i], conv_W3[i],
            conv_b3[i].reshape(1, D),
            mlp_W1[i], mlp_b1[i].reshape(1, DM), mlp_g[i].reshape(1, DM),
            mlp_beta[i].reshape(1, DM), mlp_W2[i], mlp_b2[i].reshape(1, D))

    xp = jnp.concatenate(
        [xc, jnp.zeros((NPAD - N, D), f32)], axis=0)
    bp = jnp.concatenate(
        [batch, jnp.full((NPAD - N,), NG, jnp.int32)]).reshape(1, NPAD)
    return _pool_head(xp, bp, out_W, out_b.reshape(2, 1, D),
                      out_Wf, out_bf.reshape(1, 1))
